# Initial kernel scaffold; baseline (speedup 1.0000x reference)
import jax
import jax.numpy as jnp
from jax.experimental import pallas as pl


def kernel(x, ln1_scale, ln1_bias, router_w, router_b, expert_w, proj_w, proj_b, ln2_scale, ln2_bias, mlp_w1, mlp_b1, mlp_w2, mlp_b2):
    raise NotImplementedError("write your pallas kernel here")


# speedup vs baseline: 4.5221x; 4.5221x over previous
import functools

import jax
import jax.numpy as jnp
from jax.experimental import pallas as pl
from jax.experimental.pallas import tpu as pltpu

F32 = jnp.float32
BF16 = jnp.bfloat16


def _ln(x, scale, bias, eps=1e-6):
    mu = jnp.mean(x, axis=-1, keepdims=True)
    var = jnp.mean((x - mu) ** 2, axis=-1, keepdims=True)
    return (x - mu) / jnp.sqrt(var + eps) * scale + bias


def _stage_a_kernel(x_ref, s_ref, b_ref, normed_ref, pooled_ref):
    xb = x_ref[0]
    normed = _ln(xb, s_ref[...], b_ref[...])
    normed_ref[0] = normed
    pooled_ref[0] = jnp.mean(normed, axis=0, keepdims=True)


def _stage_a2_kernel(pooled_ref, rw_ref, rb_ref,
                     idx_ref, gates_ref, aux_ref):
    B, E = pooled_ref.shape[0], rw_ref.shape[1]
    logits = jnp.dot(pooled_ref[...].astype(BF16), rw_ref[...].astype(BF16),
                     preferred_element_type=F32) + rb_ref[...]
    m = jnp.max(logits, axis=-1, keepdims=True)
    ex = jnp.exp(logits - m)
    probs = ex / jnp.sum(ex, axis=-1, keepdims=True)
    iota = jax.lax.broadcasted_iota(jnp.int32, (B, E), 1)
    v1 = jnp.max(probs, axis=-1, keepdims=True)
    i1 = jnp.min(jnp.where(probs == v1, iota, E), axis=-1, keepdims=True)
    masked = jnp.where(iota == i1, -jnp.inf, probs)
    v2 = jnp.max(masked, axis=-1, keepdims=True)
    i2 = jnp.min(jnp.where(masked == v2, iota, E), axis=-1, keepdims=True)
    s = v1 + v2
    gates_ref[...] = jnp.concatenate([v1 / s, v2 / s], axis=1)
    idx_ref[...] = jnp.concatenate([i1, i2], axis=1)
    cnt = (iota == i1).astype(F32) + (iota == i2).astype(F32)
    frac = jnp.sum(cnt, axis=0, keepdims=True) / (B * 2)
    mean_p = jnp.mean(probs, axis=0, keepdims=True)
    aux_ref[...] = E * jnp.sum(frac * mean_p, keepdims=True)


def _stage_b_kernel(idx_ref, g_ref, w_ref, x_ref, out_ref, *, H, dh):
    k = pl.program_id(1)
    i = pl.program_id(0) * 2 + k
    g = g_ref[i]
    xb = x_ref[0]
    pieces = []
    for h in range(H):
        w = w_ref[0, h].astype(BF16)
        xs = xb[:, h * dh:(h + 1) * dh].astype(BF16)
        pieces.append(jnp.dot(w, xs, preferred_element_type=F32))
    y = jnp.concatenate(pieces, axis=1) * g

    @pl.when(k == 0)
    def _():
        out_ref[0] = y

    @pl.when(k == 1)
    def _():
        out_ref[0] += y


def _stage_c_kernel(x_ref, m_ref, pw_ref, pb_ref, s2_ref, b2_ref,
                    w1_ref, b1_ref, w2_ref, b2b_ref, out_ref, *, hid_chunk):
    u = x_ref[...] + jnp.dot(m_ref[...].astype(BF16), pw_ref[...].astype(BF16),
                             preferred_element_type=F32) + pb_ref[...]
    n2 = _ln(u, s2_ref[...], b2_ref[...]).astype(BF16)
    hid = w1_ref.shape[1]
    acc = u + b2b_ref[...]
    for j in range(0, hid, hid_chunk):
        h1 = jnp.dot(n2, w1_ref[:, j:j + hid_chunk].astype(BF16),
                     preferred_element_type=F32) + b1_ref[:, j:j + hid_chunk]
        h1 = (0.5 * h1 * (1.0 + jax.lax.erf(h1 * 0.7071067811865476))).astype(BF16)
        acc = acc + jnp.dot(h1, w2_ref[j:j + hid_chunk, :].astype(BF16),
                            preferred_element_type=F32)
    out_ref[...] = acc


def kernel(x, ln1_scale, ln1_bias, router_w, router_b, expert_w, proj_w,
           proj_b, ln2_scale, ln2_bias, mlp_w1, mlp_b1, mlp_w2, mlp_b2):
    B, N, D = x.shape
    E, H = expert_w.shape[0], expert_w.shape[1]
    K = 2
    dh = D // H
    hid = mlp_w1.shape[1]

    normed, pooled = pl.pallas_call(
        _stage_a_kernel,
        grid=(B,),
        in_specs=[
            pl.BlockSpec((1, N, D), lambda b: (b, 0, 0)),
            pl.BlockSpec((1, D), lambda b: (0, 0)),
            pl.BlockSpec((1, D), lambda b: (0, 0)),
        ],
        out_specs=[
            pl.BlockSpec((1, N, D), lambda b: (b, 0, 0)),
            pl.BlockSpec((1, 1, D), lambda b: (b, 0, 0)),
        ],
        out_shape=[
            jax.ShapeDtypeStruct((B, N, D), F32),
            jax.ShapeDtypeStruct((B, 1, D), F32),
        ],
    )(x, ln1_scale.reshape(1, D), ln1_bias.reshape(1, D))
    pooled = pooled.reshape(B, D)

    top_idx, gates, aux = pl.pallas_call(
        _stage_a2_kernel,
        out_shape=[
            jax.ShapeDtypeStruct((B, K), jnp.int32),
            jax.ShapeDtypeStruct((B, K), F32),
            jax.ShapeDtypeStruct((1, 1), F32),
        ],
    )(pooled, router_w, router_b.reshape(1, E))

    mixed = pl.pallas_call(
        functools.partial(_stage_b_kernel, H=H, dh=dh),
        grid_spec=pltpu.PrefetchScalarGridSpec(
            num_scalar_prefetch=2,
            grid=(B, K),
            in_specs=[
                pl.BlockSpec((1, H, N, N),
                             lambda b, k, idx, g: (idx[b * 2 + k], 0, 0, 0)),
                pl.BlockSpec((1, N, D), lambda b, k, idx, g: (b, 0, 0)),
            ],
            out_specs=pl.BlockSpec((1, N, D), lambda b, k, idx, g: (b, 0, 0)),
        ),
        out_shape=jax.ShapeDtypeStruct((B, N, D), F32),
    )(top_idx.reshape(B * K), gates.reshape(B * K), expert_w, normed)

    R = 512
    rows = B * N
    y = pl.pallas_call(
        functools.partial(_stage_c_kernel, hid_chunk=768),
        grid=(rows // R,),
        in_specs=[
            pl.BlockSpec((R, D), lambda r: (r, 0)),
            pl.BlockSpec((R, D), lambda r: (r, 0)),
            pl.BlockSpec((D, D), lambda r: (0, 0)),
            pl.BlockSpec((1, D), lambda r: (0, 0)),
            pl.BlockSpec((1, D), lambda r: (0, 0)),
            pl.BlockSpec((1, D), lambda r: (0, 0)),
            pl.BlockSpec((D, hid), lambda r: (0, 0)),
            pl.BlockSpec((1, hid), lambda r: (0, 0)),
            pl.BlockSpec((hid, D), lambda r: (0, 0)),
            pl.BlockSpec((1, D), lambda r: (0, 0)),
        ],
        out_specs=pl.BlockSpec((R, D), lambda r: (r, 0)),
        out_shape=jax.ShapeDtypeStruct((rows, D), F32),
    )(x.reshape(rows, D), mixed.reshape(rows, D), proj_w,
      proj_b.reshape(1, D), ln2_scale.reshape(1, D), ln2_bias.reshape(1, D),
      mlp_w1, mlp_b1.reshape(1, hid), mlp_w2, mlp_b2.reshape(1, D))

    return (y.reshape(B, N, D), aux.reshape(()))


# speedup vs baseline: 5.0486x; 1.1164x over previous
import functools

import jax
import jax.numpy as jnp
from jax.experimental import pallas as pl
from jax.experimental.pallas import tpu as pltpu

F32 = jnp.float32
BF16 = jnp.bfloat16


def _ln(x, scale, bias, eps=1e-6):
    mu = jnp.mean(x, axis=-1, keepdims=True)
    var = jnp.mean((x - mu) ** 2, axis=-1, keepdims=True)
    return (x - mu) / jnp.sqrt(var + eps) * scale + bias


def _stage_a_kernel(x_ref, s_ref, b_ref, normed_ref, pooled_ref):
    xb = x_ref[0]
    normed = _ln(xb, s_ref[...], b_ref[...])
    normed_ref[0] = normed
    pooled_ref[0] = jnp.mean(normed, axis=0, keepdims=True)


def _stage_a2_kernel(pooled_ref, rw_ref, rb_ref, gmat_ref, aux_ref):
    B, E = pooled_ref.shape[0], rw_ref.shape[1]
    logits = jnp.dot(pooled_ref[...].astype(BF16), rw_ref[...].astype(BF16),
                     preferred_element_type=F32) + rb_ref[...]
    m = jnp.max(logits, axis=-1, keepdims=True)
    ex = jnp.exp(logits - m)
    probs = ex / jnp.sum(ex, axis=-1, keepdims=True)
    iota = jax.lax.broadcasted_iota(jnp.int32, (B, E), 1)
    v1 = jnp.max(probs, axis=-1, keepdims=True)
    i1 = jnp.min(jnp.where(probs == v1, iota, E), axis=-1, keepdims=True)
    masked = jnp.where(iota == i1, -jnp.inf, probs)
    v2 = jnp.max(masked, axis=-1, keepdims=True)
    i2 = jnp.min(jnp.where(masked == v2, iota, E), axis=-1, keepdims=True)
    s = v1 + v2
    gmat_ref[...] = ((iota == i1).astype(F32) * (v1 / s)
                     + (iota == i2).astype(F32) * (v2 / s))
    cnt = (iota == i1).astype(F32) + (iota == i2).astype(F32)
    frac = jnp.sum(cnt, axis=0, keepdims=True) / (B * 2)
    mean_p = jnp.mean(probs, axis=0, keepdims=True)
    aux_ref[...] = E * jnp.sum(frac * mean_p, keepdims=True)


def _stage_b_kernel(g_ref, w_ref, x_ref, out_ref, *, H, dh, B, E):
    e = pl.program_id(0)

    @pl.when(e == 0)
    def _():
        out_ref[...] = jnp.zeros_like(out_ref)

    for b in range(B):
        g = g_ref[b * E + e]

        @pl.when(g > 0.0)
        def _():
            xb = x_ref[b]
            pieces = []
            for h in range(H):
                w = w_ref[0, h].astype(BF16)
                xs = xb[:, h * dh:(h + 1) * dh].astype(BF16)
                pieces.append(jnp.dot(w, xs, preferred_element_type=F32))
            out_ref[b] += jnp.concatenate(pieces, axis=1) * g


def _stage_c_kernel(x_ref, m_ref, pw_ref, pb_ref, s2_ref, b2_ref,
                    w1_ref, b1_ref, w2_ref, b2b_ref, out_ref, *, hid_chunk):
    u = x_ref[...] + jnp.dot(m_ref[...].astype(BF16), pw_ref[...].astype(BF16),
                             preferred_element_type=F32) + pb_ref[...]
    n2 = _ln(u, s2_ref[...], b2_ref[...]).astype(BF16)
    hid = w1_ref.shape[1]
    acc = u + b2b_ref[...]
    for j in range(0, hid, hid_chunk):
        h1 = jnp.dot(n2, w1_ref[:, j:j + hid_chunk].astype(BF16),
                     preferred_element_type=F32) + b1_ref[:, j:j + hid_chunk]
        h1 = (0.5 * h1 * (1.0 + jax.lax.erf(h1 * 0.7071067811865476))).astype(BF16)
        acc = acc + jnp.dot(h1, w2_ref[j:j + hid_chunk, :].astype(BF16),
                            preferred_element_type=F32)
    out_ref[...] = acc


def kernel(x, ln1_scale, ln1_bias, router_w, router_b, expert_w, proj_w,
           proj_b, ln2_scale, ln2_bias, mlp_w1, mlp_b1, mlp_w2, mlp_b2):
    B, N, D = x.shape
    E, H = expert_w.shape[0], expert_w.shape[1]
    K = 2
    dh = D // H
    hid = mlp_w1.shape[1]

    normed, pooled = pl.pallas_call(
        _stage_a_kernel,
        grid=(B,),
        in_specs=[
            pl.BlockSpec((1, N, D), lambda b: (b, 0, 0)),
            pl.BlockSpec((1, D), lambda b: (0, 0)),
            pl.BlockSpec((1, D), lambda b: (0, 0)),
        ],
        out_specs=[
            pl.BlockSpec((1, N, D), lambda b: (b, 0, 0)),
            pl.BlockSpec((1, 1, D), lambda b: (b, 0, 0)),
        ],
        out_shape=[
            jax.ShapeDtypeStruct((B, N, D), F32),
            jax.ShapeDtypeStruct((B, 1, D), F32),
        ],
    )(x, ln1_scale.reshape(1, D), ln1_bias.reshape(1, D))
    pooled = pooled.reshape(B, D)

    gmat, aux = pl.pallas_call(
        _stage_a2_kernel,
        out_shape=[
            jax.ShapeDtypeStruct((B, E), F32),
            jax.ShapeDtypeStruct((1, 1), F32),
        ],
    )(pooled, router_w, router_b.reshape(1, E))

    mixed = pl.pallas_call(
        functools.partial(_stage_b_kernel, H=H, dh=dh, B=B, E=E),
        grid_spec=pltpu.PrefetchScalarGridSpec(
            num_scalar_prefetch=1,
            grid=(E,),
            in_specs=[
                pl.BlockSpec((1, H, N, N), lambda e, g: (e, 0, 0, 0)),
                pl.BlockSpec((B, N, D), lambda e, g: (0, 0, 0)),
            ],
            out_specs=pl.BlockSpec((B, N, D), lambda e, g: (0, 0, 0)),
        ),
        out_shape=jax.ShapeDtypeStruct((B, N, D), F32),
    )(gmat.reshape(B * E), expert_w, normed)

    R = 512
    rows = B * N
    y = pl.pallas_call(
        functools.partial(_stage_c_kernel, hid_chunk=768),
        grid=(rows // R,),
        in_specs=[
            pl.BlockSpec((R, D), lambda r: (r, 0)),
            pl.BlockSpec((R, D), lambda r: (r, 0)),
            pl.BlockSpec((D, D), lambda r: (0, 0)),
            pl.BlockSpec((1, D), lambda r: (0, 0)),
            pl.BlockSpec((1, D), lambda r: (0, 0)),
            pl.BlockSpec((1, D), lambda r: (0, 0)),
            pl.BlockSpec((D, hid), lambda r: (0, 0)),
            pl.BlockSpec((1, hid), lambda r: (0, 0)),
            pl.BlockSpec((hid, D), lambda r: (0, 0)),
            pl.BlockSpec((1, D), lambda r: (0, 0)),
        ],
        out_specs=pl.BlockSpec((R, D), lambda r: (r, 0)),
        out_shape=jax.ShapeDtypeStruct((rows, D), F32),
    )(x.reshape(rows, D), mixed.reshape(rows, D), proj_w,
      proj_b.reshape(1, D), ln2_scale.reshape(1, D), ln2_bias.reshape(1, D),
      mlp_w1, mlp_b1.reshape(1, hid), mlp_w2, mlp_b2.reshape(1, D))

    return (y.reshape(B, N, D), aux.reshape(()))
